# baseline (speedup 1.0000x reference)
import functools

import jax
import jax.numpy as jnp
from jax import lax
from jax.experimental import pallas as pl
from jax.experimental.pallas import tpu as pltpu
from jax.experimental.pallas import tpu_sc as plsc

FEATURE_DIM = 2048
HIDDEN_DIM = 1024
NUM_EXPERTS = 64
TOP_K = 8
TEMPERATURE = 1.0
BATCH = 8192

BLOCK_M = 2048
SUB_M = 256


def _scores_body(x, w1, b1, w2, b2, en_n, ts):
    h = lax.dot_general(x, w1, (((1,), (1,)), ((), ())),
                        preferred_element_type=jnp.float32)
    h = jnp.maximum(h + b1, 0.0)
    logits = lax.dot_general(h, w2, (((1,), (1,)), ((), ())),
                             preferred_element_type=jnp.float32)
    logits = (logits + b2) / TEMPERATURE
    m = jnp.max(logits, axis=1, keepdims=True)
    e = jnp.exp(logits - m)
    probs = e / jnp.sum(e, axis=1, keepdims=True)
    xn = x / (jnp.sqrt(jnp.sum(x * x, axis=1, keepdims=True)) + 1e-8)
    raw = lax.dot_general(xn, en_n, (((1,), (1,)), ((), ())),
                          preferred_element_type=jnp.float32)
    sim = (raw + 1.0) * 0.5
    return probs * sim * ts


def _scores_kernel(x_ref, w1_ref, b1_ref, w2_ref, b2_ref, en_ref,
                   trust_ref, stale_ref, s_out_ref):
    w1 = w1_ref[...]
    b1 = b1_ref[...]
    w2 = w2_ref[...]
    b2 = b2_ref[...]
    en = en_ref[...]
    en_n = en / (jnp.sqrt(jnp.sum(en * en, axis=1, keepdims=True)) + 1e-8)
    ts = trust_ref[...] * stale_ref[...]
    for s in range(BLOCK_M // SUB_M):
        x = x_ref[s * SUB_M:(s + 1) * SUB_M, :]
        scores = _scores_body(x, w1, b1, w2, b2, en_n, ts)
        s_out_ref[:, s * SUB_M:(s + 1) * SUB_M] = scores.T


def _tc_scores(x, W1, b1, W2, b2, expert_features, trust, staleness):
    grid = (BATCH // BLOCK_M,)
    fixed = lambda i: (0, 0)
    return pl.pallas_call(
        _scores_kernel,
        grid=grid,
        in_specs=[
            pl.BlockSpec((BLOCK_M, FEATURE_DIM), lambda i: (i, 0)),
            pl.BlockSpec((HIDDEN_DIM, FEATURE_DIM), fixed),
            pl.BlockSpec((1, HIDDEN_DIM), fixed),
            pl.BlockSpec((NUM_EXPERTS, HIDDEN_DIM), fixed),
            pl.BlockSpec((1, NUM_EXPERTS), fixed),
            pl.BlockSpec((NUM_EXPERTS, FEATURE_DIM), fixed),
            pl.BlockSpec((1, NUM_EXPERTS), fixed),
            pl.BlockSpec((1, NUM_EXPERTS), fixed),
        ],
        out_specs=pl.BlockSpec((NUM_EXPERTS, BLOCK_M), lambda i: (0, i)),
        out_shape=jax.ShapeDtypeStruct((NUM_EXPERTS, BATCH), jnp.float32),
        compiler_params=pltpu.CompilerParams(
            dimension_semantics=("parallel",)),
    )(x, W1, b1.reshape(1, -1), W2, b2.reshape(1, -1),
      expert_features, trust.reshape(1, -1), staleness.reshape(1, -1))


def _make_sc_topk():
    info = plsc.get_sparse_core_info()
    nw = info.num_cores * info.num_subcores
    cols = BATCH // nw
    groups = cols // 16

    mesh = plsc.VectorSubcoreMesh(core_axis_name="c", subcore_axis_name="s")

    @functools.partial(
        pl.kernel, mesh=mesh,
        out_type=[
            jax.ShapeDtypeStruct((TOP_K, BATCH), jnp.float32),
            jax.ShapeDtypeStruct((TOP_K, BATCH), jnp.int32),
        ],
        scratch_types=[
            pltpu.VMEM((NUM_EXPERTS, cols), jnp.float32),
            pltpu.VMEM((TOP_K, cols), jnp.float32),
            pltpu.VMEM((TOP_K, cols), jnp.int32),
        ],
    )
    def sc_topk(s_hbm, w_hbm, i_hbm, chunk_v, wv, iv):
        wid = lax.axis_index("s") * info.num_cores + lax.axis_index("c")
        base = wid * cols
        pltpu.sync_copy(s_hbm.at[:, pl.ds(base, cols)], chunk_v)

        def body(g, _):
            off = g * 16
            vals = [jnp.full((16,), -jnp.inf, jnp.float32)
                    for _ in range(TOP_K)]
            idxs = [jnp.zeros((16,), jnp.int32) for _ in range(TOP_K)]
            for e in range(NUM_EXPERTS):
                v = chunk_v[e, pl.ds(off, 16)]
                ei = jnp.full((16,), e, jnp.int32)
                for j in range(TOP_K):
                    c = v > vals[j]
                    nv = jnp.where(c, v, vals[j])
                    v = jnp.where(c, vals[j], v)
                    vals[j] = nv
                    ni = jnp.where(c, ei, idxs[j])
                    ei = jnp.where(c, idxs[j], ei)
                    idxs[j] = ni
            tot = vals[0]
            for j in range(1, TOP_K):
                tot = tot + vals[j]
            inv = 1.0 / (tot + 1e-9)
            for j in range(TOP_K):
                wv[j, pl.ds(off, 16)] = vals[j] * inv
                iv[j, pl.ds(off, 16)] = idxs[j]
            return _

        lax.fori_loop(0, groups, body, None)
        pltpu.sync_copy(wv, w_hbm.at[:, pl.ds(base, cols)])
        pltpu.sync_copy(iv, i_hbm.at[:, pl.ds(base, cols)])

    return sc_topk


_sc_topk = _make_sc_topk()


@jax.jit
def kernel(x, W1, b1, W2, b2, expert_features, trust, staleness):
    scores_t = _tc_scores(x, W1, b1, W2, b2, expert_features, trust,
                          staleness)
    w_t, i_t = _sc_topk(scores_t)
    return w_t.T, i_t.T


# speedup vs baseline: 1.1298x; 1.1298x over previous
import jax
import jax.numpy as jnp
from jax import lax
from jax.experimental import pallas as pl
from jax.experimental.pallas import tpu as pltpu

FEATURE_DIM = 2048
HIDDEN_DIM = 1024
NUM_EXPERTS = 64
TOP_K = 8
TEMPERATURE = 1.0
BATCH = 8192

BLOCK_M = 2048
SUB_M = 256


def _score_and_select(x, w1, b1, w2, b2, en_n, ts):
    h = lax.dot_general(x, w1, (((1,), (1,)), ((), ())),
                        preferred_element_type=jnp.float32)
    h = jnp.maximum(h + b1, 0.0)
    logits = lax.dot_general(h, w2, (((1,), (1,)), ((), ())),
                             preferred_element_type=jnp.float32)
    logits = (logits + b2) / TEMPERATURE
    m = jnp.max(logits, axis=1, keepdims=True)
    e = jnp.exp(logits - m)
    probs = e / jnp.sum(e, axis=1, keepdims=True)
    xn = x / (jnp.sqrt(jnp.sum(x * x, axis=1, keepdims=True)) + 1e-8)
    raw = lax.dot_general(xn, en_n, (((1,), (1,)), ((), ())),
                          preferred_element_type=jnp.float32)
    sim = (raw + 1.0) * 0.5
    scores = probs * sim * ts
    cols = lax.broadcasted_iota(jnp.int32, scores.shape, 1)
    vals = []
    idxs = []
    for _ in range(TOP_K):
        v = jnp.max(scores, axis=1)
        i = jnp.argmax(scores, axis=1).astype(jnp.int32)
        vals.append(v)
        idxs.append(i)
        scores = jnp.where(cols == i[:, None], -jnp.inf, scores)
    topv = jnp.stack(vals, axis=1)
    topi = jnp.stack(idxs, axis=1)
    topw = topv / (jnp.sum(topv, axis=1, keepdims=True) + 1e-9)
    return jnp.concatenate(
        [topw, lax.bitcast_convert_type(topi, jnp.float32)], axis=1)


def _router_kernel(x_ref, w1_ref, b1_ref, w2_ref, b2_ref, en_ref,
                   trust_ref, stale_ref, out_ref):
    w1 = w1_ref[...]
    b1 = b1_ref[...]
    w2 = w2_ref[...]
    b2 = b2_ref[...]
    en = en_ref[...]
    en_n = en / (jnp.sqrt(jnp.sum(en * en, axis=1, keepdims=True)) + 1e-8)
    ts = trust_ref[...] * stale_ref[...]
    for s in range(BLOCK_M // SUB_M):
        x = x_ref[s * SUB_M:(s + 1) * SUB_M, :]
        packed = _score_and_select(x, w1, b1, w2, b2, en_n, ts)
        out_ref[s * SUB_M:(s + 1) * SUB_M, :] = packed


@jax.jit
def kernel(x, W1, b1, W2, b2, expert_features, trust, staleness):
    grid = (BATCH // BLOCK_M,)
    fixed = lambda i: (0, 0)
    out = pl.pallas_call(
        _router_kernel,
        grid=grid,
        in_specs=[
            pl.BlockSpec((BLOCK_M, FEATURE_DIM), lambda i: (i, 0)),
            pl.BlockSpec((HIDDEN_DIM, FEATURE_DIM), fixed),
            pl.BlockSpec((1, HIDDEN_DIM), fixed),
            pl.BlockSpec((NUM_EXPERTS, HIDDEN_DIM), fixed),
            pl.BlockSpec((1, NUM_EXPERTS), fixed),
            pl.BlockSpec((NUM_EXPERTS, FEATURE_DIM), fixed),
            pl.BlockSpec((1, NUM_EXPERTS), fixed),
            pl.BlockSpec((1, NUM_EXPERTS), fixed),
        ],
        out_specs=pl.BlockSpec((BLOCK_M, 2 * TOP_K), lambda i: (i, 0)),
        out_shape=jax.ShapeDtypeStruct((BATCH, 2 * TOP_K), jnp.float32),
        compiler_params=pltpu.CompilerParams(
            dimension_semantics=("parallel",)),
    )(x, W1, b1.reshape(1, -1), W2, b2.reshape(1, -1),
      expert_features, trust.reshape(1, -1), staleness.reshape(1, -1))
    return out[:, :TOP_K], lax.bitcast_convert_type(out[:, TOP_K:],
                                                    jnp.int32)


# speedup vs baseline: 1.1772x; 1.0420x over previous
import jax
import jax.numpy as jnp
from jax import lax
from jax.experimental import pallas as pl
from jax.experimental.pallas import tpu as pltpu

FEATURE_DIM = 2048
HIDDEN_DIM = 1024
NUM_EXPERTS = 64
TOP_K = 8
TEMPERATURE = 1.0
BATCH = 8192

BLOCK_M = 2048
SUB_M = 256


def _score_and_select(x, w1, b1, w2, b2, en_n, ts):
    h = lax.dot_general(x, w1, (((1,), (1,)), ((), ())),
                        preferred_element_type=jnp.float32)
    h = jnp.maximum(h + b1, 0.0)
    logits = lax.dot_general(h, w2, (((1,), (1,)), ((), ())),
                             preferred_element_type=jnp.float32)
    logits = (logits + b2) / TEMPERATURE
    m = jnp.max(logits, axis=1, keepdims=True)
    e = jnp.exp(logits - m)
    probs = e / jnp.sum(e, axis=1, keepdims=True)
    xn = x / (jnp.sqrt(jnp.sum(x * x, axis=1, keepdims=True)) + 1e-8)
    raw = lax.dot_general(xn, en_n, (((1,), (1,)), ((), ())),
                          preferred_element_type=jnp.float32)
    sim = (raw + 1.0) * 0.5
    scores = probs * sim * ts
    cols = lax.broadcasted_iota(jnp.int32, scores.shape, 1)
    vals = []
    idxs = []
    for _ in range(TOP_K):
        v = jnp.max(scores, axis=1)
        i = jnp.argmax(scores, axis=1).astype(jnp.int32)
        vals.append(v)
        idxs.append(i)
        scores = jnp.where(cols == i[:, None], -jnp.inf, scores)
    topv = jnp.stack(vals, axis=1)
    topi = jnp.stack(idxs, axis=1)
    topw = topv / (jnp.sum(topv, axis=1, keepdims=True) + 1e-9)
    return topw, topi


def _router_kernel(x_ref, w1_ref, b1_ref, w2_ref, b2_ref, en_ref,
                   trust_ref, stale_ref, w_out_ref, i_out_ref):
    w1 = w1_ref[...]
    b1 = b1_ref[...]
    w2 = w2_ref[...]
    b2 = b2_ref[...]
    en = en_ref[...]
    en_n = en / (jnp.sqrt(jnp.sum(en * en, axis=1, keepdims=True)) + 1e-8)
    ts = trust_ref[...] * stale_ref[...]
    for s in range(BLOCK_M // SUB_M):
        x = x_ref[s * SUB_M:(s + 1) * SUB_M, :]
        topw, topi = _score_and_select(x, w1, b1, w2, b2, en_n, ts)
        w_out_ref[s * SUB_M:(s + 1) * SUB_M, :] = topw
        i_out_ref[s * SUB_M:(s + 1) * SUB_M, :] = topi


@jax.jit
def kernel(x, W1, b1, W2, b2, expert_features, trust, staleness):
    grid = (BATCH // BLOCK_M,)
    fixed = lambda i: (0, 0)
    out = pl.pallas_call(
        _router_kernel,
        grid=grid,
        in_specs=[
            pl.BlockSpec((BLOCK_M, FEATURE_DIM), lambda i: (i, 0)),
            pl.BlockSpec((HIDDEN_DIM, FEATURE_DIM), fixed),
            pl.BlockSpec((1, HIDDEN_DIM), fixed),
            pl.BlockSpec((NUM_EXPERTS, HIDDEN_DIM), fixed),
            pl.BlockSpec((1, NUM_EXPERTS), fixed),
            pl.BlockSpec((NUM_EXPERTS, FEATURE_DIM), fixed),
            pl.BlockSpec((1, NUM_EXPERTS), fixed),
            pl.BlockSpec((1, NUM_EXPERTS), fixed),
        ],
        out_specs=[
            pl.BlockSpec((BLOCK_M, TOP_K), lambda i: (i, 0)),
            pl.BlockSpec((BLOCK_M, TOP_K), lambda i: (i, 0)),
        ],
        out_shape=[
            jax.ShapeDtypeStruct((BATCH, TOP_K), jnp.float32),
            jax.ShapeDtypeStruct((BATCH, TOP_K), jnp.int32),
        ],
        compiler_params=pltpu.CompilerParams(
            dimension_semantics=("parallel",)),
    )(x, W1, b1.reshape(1, -1), W2, b2.reshape(1, -1),
      expert_features, trust.reshape(1, -1), staleness.reshape(1, -1))
    return out[0], out[1]
